# baseline (speedup 1.0000x reference)
import jax
import jax.numpy as jnp
from jax.experimental import pallas as pl
from jax.experimental.pallas import tpu as pltpu


def kernel(x, table):
    B, S = x.shape
    M, D = table.shape
    TILE = 1024
    ntiles = S // TILE

    def body(tab_ref, out_ref, sem):
        i = pl.program_id(0)
        copies = [
            pltpu.make_async_copy(
                tab_ref,
                out_ref.at[b, pl.ds(i * TILE, TILE), :],
                sem.at[b],
            )
            for b in range(B)
        ]
        for c in copies:
            c.start()
        for c in copies:
            c.wait()

    out = pl.pallas_call(
        body,
        grid=(ntiles,),
        in_specs=[pl.BlockSpec((TILE, D), lambda i: (i, 0))],
        out_specs=pl.BlockSpec(memory_space=pltpu.MemorySpace.HBM),
        out_shape=jax.ShapeDtypeStruct((B, S, D), table.dtype),
        scratch_shapes=[pltpu.SemaphoreType.DMA((B,))],
    )(table)
    return out


# speedup vs baseline: 1.0114x; 1.0114x over previous
import jax
import jax.numpy as jnp
from jax.experimental import pallas as pl
from jax.experimental.pallas import tpu as pltpu


def kernel(x, table):
    B, S = x.shape
    M, D = table.shape
    TILE = 1024
    N = S // TILE

    def body(tab_hbm, out_hbm, buf, in_sem, out_sem):
        def in_copy(i, p):
            return pltpu.make_async_copy(
                tab_hbm.at[pl.ds(i * TILE, TILE), :], buf.at[p], in_sem.at[p]
            )

        def out_copy(i, p, b):
            return pltpu.make_async_copy(
                buf.at[p], out_hbm.at[b, pl.ds(i * TILE, TILE), :], out_sem.at[p]
            )

        in_copy(0, 0).start()
        for i in range(N):
            p = i % 2
            if i + 1 < N:
                if i >= 1:
                    for b in range(B):
                        out_copy(i - 1, p ^ 1, b).wait()
                in_copy(i + 1, p ^ 1).start()
            in_copy(i, p).wait()
            for b in range(B):
                out_copy(i, p, b).start()
        for i in (N - 2, N - 1):
            for b in range(B):
                out_copy(i, i % 2, b).wait()

    out = pl.pallas_call(
        body,
        in_specs=[pl.BlockSpec(memory_space=pltpu.MemorySpace.HBM)],
        out_specs=pl.BlockSpec(memory_space=pltpu.MemorySpace.HBM),
        out_shape=jax.ShapeDtypeStruct((B, S, D), table.dtype),
        scratch_shapes=[
            pltpu.VMEM((2, TILE, D), table.dtype),
            pltpu.SemaphoreType.DMA((2,)),
            pltpu.SemaphoreType.DMA((2,)),
        ],
    )(table)
    return out


# speedup vs baseline: 1.0656x; 1.0537x over previous
import jax
import jax.numpy as jnp
from jax.experimental import pallas as pl
from jax.experimental.pallas import tpu as pltpu


def kernel(x, table):
    B, S = x.shape
    M, D = table.shape
    TILE = 2048
    N = S // TILE

    def body(tab_hbm, out_hbm, buf, in_sem, out_sem):
        def in_copy(i, p):
            return pltpu.make_async_copy(
                tab_hbm.at[pl.ds(i * TILE, TILE), :], buf.at[p], in_sem.at[p]
            )

        def out_copy(i, p, b):
            return pltpu.make_async_copy(
                buf.at[p], out_hbm.at[b, pl.ds(i * TILE, TILE), :], out_sem.at[p]
            )

        in_copy(0, 0).start()
        for i in range(N):
            p = i % 2
            if i + 1 < N:
                if i >= 1:
                    for b in range(B):
                        out_copy(i - 1, p ^ 1, b).wait()
                in_copy(i + 1, p ^ 1).start()
            in_copy(i, p).wait()
            for b in range(B):
                out_copy(i, p, b).start()
        for i in (N - 2, N - 1):
            for b in range(B):
                out_copy(i, i % 2, b).wait()

    out = pl.pallas_call(
        body,
        in_specs=[pl.BlockSpec(memory_space=pltpu.MemorySpace.HBM)],
        out_specs=pl.BlockSpec(memory_space=pltpu.MemorySpace.HBM),
        out_shape=jax.ShapeDtypeStruct((B, S, D), table.dtype),
        scratch_shapes=[
            pltpu.VMEM((2, TILE, D), table.dtype),
            pltpu.SemaphoreType.DMA((2,)),
            pltpu.SemaphoreType.DMA((2,)),
        ],
    )(table)
    return out


# speedup vs baseline: 1.1021x; 1.0342x over previous
import jax
import jax.numpy as jnp
from jax.experimental import pallas as pl
from jax.experimental.pallas import tpu as pltpu


def kernel(x, table):
    B, S = x.shape
    M, D = table.shape
    TILE = 4096
    N = S // TILE

    def body(tab_hbm, out_hbm, buf, in_sem, out_sem):
        def in_copy(i, p):
            return pltpu.make_async_copy(
                tab_hbm.at[pl.ds(i * TILE, TILE), :], buf.at[p], in_sem.at[p]
            )

        def out_copy(i, p, b):
            return pltpu.make_async_copy(
                buf.at[p], out_hbm.at[b, pl.ds(i * TILE, TILE), :], out_sem.at[p]
            )

        in_copy(0, 0).start()
        for i in range(N):
            p = i % 2
            if i + 1 < N:
                if i >= 1:
                    for b in range(B):
                        out_copy(i - 1, p ^ 1, b).wait()
                in_copy(i + 1, p ^ 1).start()
            in_copy(i, p).wait()
            for b in range(B):
                out_copy(i, p, b).start()
        for i in (N - 2, N - 1):
            for b in range(B):
                out_copy(i, i % 2, b).wait()

    out = pl.pallas_call(
        body,
        in_specs=[pl.BlockSpec(memory_space=pltpu.MemorySpace.HBM)],
        out_specs=pl.BlockSpec(memory_space=pltpu.MemorySpace.HBM),
        out_shape=jax.ShapeDtypeStruct((B, S, D), table.dtype),
        scratch_shapes=[
            pltpu.VMEM((2, TILE, D), table.dtype),
            pltpu.SemaphoreType.DMA((2,)),
            pltpu.SemaphoreType.DMA((2,)),
        ],
    )(table)
    return out
